# Initial kernel scaffold; baseline (speedup 1.0000x reference)
import jax
import jax.numpy as jnp
from jax.experimental import pallas as pl


def kernel(X, indices, T, proxies):
    raise NotImplementedError("write your pallas kernel here")


# speedup vs baseline: 4.9192x; 4.9192x over previous
import functools

import jax
import jax.numpy as jnp
from jax.experimental import pallas as pl
from jax.experimental.pallas import tpu as pltpu

NB = 8192
EMB = 64
BATCH = 1024
BB = 256
SCALE = 3.0


def _loss_kernel(x_ref, t_ref, p_ref, out_ref, pn_ref, sqp_ref):
    i = pl.program_id(0)

    @pl.when(i == 0)
    def _():
        P = p_ref[...]
        sq = jnp.sum(P * P, axis=1, keepdims=True)
        Pn = P * (SCALE / jnp.sqrt(sq + 1e-12))
        pn_ref[...] = Pn
        sqp_ref[...] = jnp.sum(Pn * Pn, axis=1, keepdims=True).reshape(1, NB)
        out_ref[0, 0] = 0.0

    X = x_ref[...]
    sqx = jnp.sum(X * X, axis=1, keepdims=True)
    Xn = X * (SCALE / jnp.sqrt(sqx + 1e-12))
    sqxn = jnp.sum(Xn * Xn, axis=1, keepdims=True)

    ip = jax.lax.dot_general(
        Xn, pn_ref[...], (((1,), (1,)), ((), ())),
        preferred_element_type=jnp.float32)
    d = jnp.maximum(sqxn + sqp_ref[...] - 2.0 * ip, 0.0)

    neg = -d
    m = jnp.max(neg, axis=1, keepdims=True)
    lse = m[:, 0] + jnp.log(jnp.sum(jnp.exp(neg - m), axis=1))

    t = t_ref[0, 0, :]
    cols = jax.lax.broadcasted_iota(jnp.int32, (BB, NB), 1)
    d_t = jnp.sum(jnp.where(cols == t[:, None], d, 0.0), axis=1)

    out_ref[0, 0] += jnp.sum(d_t + lse) * (1.0 / BATCH)


@functools.partial(jax.jit, static_argnames=())
def kernel(X, indices, T, proxies):
    del indices
    nblk = BATCH // BB
    t3 = T.reshape(nblk, 1, BB)
    out = pl.pallas_call(
        _loss_kernel,
        grid=(nblk,),
        in_specs=[
            pl.BlockSpec((BB, EMB), lambda i: (i, 0)),
            pl.BlockSpec((1, 1, BB), lambda i: (i, 0, 0)),
            pl.BlockSpec((NB, EMB), lambda i: (0, 0)),
        ],
        out_specs=pl.BlockSpec((1, 1), lambda i: (0, 0),
                               memory_space=pltpu.SMEM),
        out_shape=jax.ShapeDtypeStruct((1, 1), jnp.float32),
        scratch_shapes=[
            pltpu.VMEM((NB, EMB), jnp.float32),
            pltpu.VMEM((1, NB), jnp.float32),
        ],
        compiler_params=pltpu.CompilerParams(
            dimension_semantics=("arbitrary",)),
    )(X, t3, proxies)
    return out[0, 0]


# speedup vs baseline: 5.5545x; 1.1292x over previous
import functools

import jax
import jax.numpy as jnp
from jax.experimental import pallas as pl
from jax.experimental.pallas import tpu as pltpu

NB = 8192
EMB = 64
BATCH = 1024
BB = 256
SCALE = 3.0


def _loss_kernel(x_ref, t_ref, p_ref, out_ref, pn_ref, sqp_ref):
    i = pl.program_id(0)

    @pl.when(i == 0)
    def _():
        P = p_ref[...]
        sq = jnp.sum(P * P, axis=1, keepdims=True)
        Pn = P * (SCALE / jnp.sqrt(sq + 1e-12))
        pn_ref[...] = Pn
        sqp_ref[...] = jnp.sum(Pn * Pn, axis=1, keepdims=True).reshape(1, NB)
        out_ref[0, 0] = 0.0

    X = x_ref[...]
    sqx = jnp.sum(X * X, axis=1, keepdims=True)
    Xn = X * (SCALE / jnp.sqrt(sqx + 1e-12))
    sqxn = jnp.sum(Xn * Xn, axis=1, keepdims=True)

    ip = jax.lax.dot_general(
        Xn, pn_ref[...], (((1,), (1,)), ((), ())),
        preferred_element_type=jnp.float32)
    d = jnp.maximum(sqxn + sqp_ref[...] - 2.0 * ip, 0.0)

    e = jnp.exp(-d)
    lse = jnp.log(jnp.sum(e, axis=1))

    t = t_ref[0, 0, :]
    cols = jax.lax.broadcasted_iota(jnp.int32, (BB, NB), 1)
    d_t = jnp.sum(jnp.where(cols == t[:, None], d, 0.0), axis=1)

    out_ref[0, 0] += jnp.sum(d_t + lse) * (1.0 / BATCH)


@functools.partial(jax.jit, static_argnames=())
def kernel(X, indices, T, proxies):
    del indices
    nblk = BATCH // BB
    t3 = T.reshape(nblk, 1, BB)
    out = pl.pallas_call(
        _loss_kernel,
        grid=(nblk,),
        in_specs=[
            pl.BlockSpec((BB, EMB), lambda i: (i, 0)),
            pl.BlockSpec((1, 1, BB), lambda i: (i, 0, 0)),
            pl.BlockSpec((NB, EMB), lambda i: (0, 0)),
        ],
        out_specs=pl.BlockSpec((1, 1), lambda i: (0, 0),
                               memory_space=pltpu.SMEM),
        out_shape=jax.ShapeDtypeStruct((1, 1), jnp.float32),
        scratch_shapes=[
            pltpu.VMEM((NB, EMB), jnp.float32),
            pltpu.VMEM((1, NB), jnp.float32),
        ],
        compiler_params=pltpu.CompilerParams(
            dimension_semantics=("arbitrary",)),
    )(X, t3, proxies)
    return out[0, 0]


# speedup vs baseline: 6.7365x; 1.2128x over previous
import functools

import jax
import jax.numpy as jnp
from jax.experimental import pallas as pl
from jax.experimental.pallas import tpu as pltpu

NB = 8192
EMB = 64
KAUG = 128
BATCH = 1024
BB = 256
PB = 2048
SCALE = 3.0


def _prep_kernel(p_ref, paug_ref):
    P = p_ref[...]
    sq = jnp.sum(P * P, axis=1, keepdims=True)
    Pn = P * (SCALE / jnp.sqrt(sq + 1e-12))
    nsqp = -jnp.sum(Pn * Pn, axis=1, keepdims=True)
    paug_ref[...] = jnp.concatenate(
        [Pn, nsqp, jnp.zeros((PB, KAUG - EMB - 1), jnp.float32)], axis=1)


def _loss_kernel(x_ref, t_ref, paug_ref, out_ref):
    i = pl.program_id(0)

    @pl.when(i == 0)
    def _():
        out_ref[0, 0] = 0.0

    X = x_ref[...]
    sqx = jnp.sum(X * X, axis=1, keepdims=True)
    x2 = X * ((2.0 * SCALE) / jnp.sqrt(sqx + 1e-12))
    xaug = jnp.concatenate(
        [x2, jnp.ones((BB, 1), jnp.float32),
         jnp.zeros((BB, KAUG - EMB - 1), jnp.float32)], axis=1)

    m = jax.lax.dot_general(
        xaug, paug_ref[...], (((1,), (1,)), ((), ())),
        preferred_element_type=jnp.float32)

    s = jnp.sum(jnp.exp(m), axis=1)

    t = t_ref[0, 0, :]
    cols = jax.lax.broadcasted_iota(jnp.int32, (BB, NB), 1)
    m_t = jnp.sum(jnp.where(cols == t[:, None], m, 0.0), axis=1)

    out_ref[0, 0] += jnp.sum(jnp.log(s) - m_t) * (1.0 / BATCH)


@functools.partial(jax.jit, static_argnames=())
def kernel(X, indices, T, proxies):
    del indices
    paug = pl.pallas_call(
        _prep_kernel,
        grid=(NB // PB,),
        in_specs=[pl.BlockSpec((PB, EMB), lambda i: (i, 0))],
        out_specs=pl.BlockSpec((PB, KAUG), lambda i: (i, 0)),
        out_shape=jax.ShapeDtypeStruct((NB, KAUG), jnp.float32),
        compiler_params=pltpu.CompilerParams(
            dimension_semantics=("arbitrary",)),
    )(proxies)

    nblk = BATCH // BB
    t3 = T.reshape(nblk, 1, BB)
    out = pl.pallas_call(
        _loss_kernel,
        grid=(nblk,),
        in_specs=[
            pl.BlockSpec((BB, EMB), lambda i: (i, 0)),
            pl.BlockSpec((1, 1, BB), lambda i: (i, 0, 0)),
            pl.BlockSpec((NB, KAUG), lambda i: (0, 0)),
        ],
        out_specs=pl.BlockSpec((1, 1), lambda i: (0, 0),
                               memory_space=pltpu.SMEM),
        out_shape=jax.ShapeDtypeStruct((1, 1), jnp.float32),
        compiler_params=pltpu.CompilerParams(
            dimension_semantics=("arbitrary",)),
    )(X, t3, paug)
    return out[0, 0]


# speedup vs baseline: 7.5979x; 1.1279x over previous
import functools

import jax
import jax.numpy as jnp
from jax.experimental import pallas as pl
from jax.experimental.pallas import tpu as pltpu

NB = 8192
EMB = 64
KAUG = 128
BATCH = 1024
PB = 2048
NSTEP = NB // PB
SCALE = 3.0


def _loss_kernel(x_ref, t_ref, p_ref, out_ref, s_ref, mt_ref):
    j = pl.program_id(0)

    @pl.when(j == 0)
    def _():
        s_ref[...] = jnp.zeros_like(s_ref)
        mt_ref[...] = jnp.zeros_like(mt_ref)

    X = x_ref[...]
    sqx = jnp.sum(X * X, axis=1, keepdims=True)
    x2 = X * ((2.0 * SCALE) / jnp.sqrt(sqx + 1e-12))
    xaug = jnp.concatenate(
        [x2, jnp.ones((BATCH, 1), jnp.float32),
         jnp.zeros((BATCH, KAUG - EMB - 1), jnp.float32)],
        axis=1).astype(jnp.bfloat16)

    P = p_ref[...]
    sqp = jnp.sum(P * P, axis=1, keepdims=True)
    Pn = P * (SCALE / jnp.sqrt(sqp + 1e-12))
    nsqpn = -jnp.sum(Pn * Pn, axis=1, keepdims=True)
    paug = jnp.concatenate(
        [Pn, nsqpn, jnp.zeros((PB, KAUG - EMB - 1), jnp.float32)],
        axis=1).astype(jnp.bfloat16)

    m = jax.lax.dot_general(
        xaug, paug, (((1,), (1,)), ((), ())),
        preferred_element_type=jnp.float32)

    s_ref[...] += jnp.sum(jnp.exp(m), axis=1, keepdims=True)

    t = t_ref[...]
    cols = j * PB + jax.lax.broadcasted_iota(jnp.int32, (BATCH, PB), 1)
    mt_ref[...] += jnp.sum(jnp.where(cols == t, m, 0.0), axis=1,
                           keepdims=True)

    @pl.when(j == NSTEP - 1)
    def _():
        out_ref[0, 0] = jnp.sum(jnp.log(s_ref[...]) - mt_ref[...]) * (
            1.0 / BATCH)


@functools.partial(jax.jit, static_argnames=())
def kernel(X, indices, T, proxies):
    del indices
    t2 = T.reshape(BATCH, 1)
    out = pl.pallas_call(
        _loss_kernel,
        grid=(NSTEP,),
        in_specs=[
            pl.BlockSpec((BATCH, EMB), lambda j: (0, 0)),
            pl.BlockSpec((BATCH, 1), lambda j: (0, 0)),
            pl.BlockSpec((PB, EMB), lambda j: (j, 0)),
        ],
        out_specs=pl.BlockSpec((1, 1), lambda j: (0, 0),
                               memory_space=pltpu.SMEM),
        out_shape=jax.ShapeDtypeStruct((1, 1), jnp.float32),
        scratch_shapes=[
            pltpu.VMEM((BATCH, 1), jnp.float32),
            pltpu.VMEM((BATCH, 1), jnp.float32),
        ],
        compiler_params=pltpu.CompilerParams(
            dimension_semantics=("arbitrary",)),
    )(X, t2, proxies)
    return out[0, 0]
